# Initial kernel scaffold; baseline (speedup 1.0000x reference)
import jax
import jax.numpy as jnp
from jax.experimental import pallas as pl


def kernel(features, edge_index, W1, b1, W2, b2):
    raise NotImplementedError("write your pallas kernel here")


# speedup vs baseline: 4.9193x; 4.9193x over previous
import functools

import jax
import jax.numpy as jnp
from jax import lax
from jax.experimental import pallas as pl
from jax.experimental.pallas import tpu as pltpu
from jax.experimental.pallas import tpu_sc as plsc

N = 10000
E = 320000
D = 128
NC = 2
NS = 16
NW = NC * NS
EPT = E // NW
CH = 128
NCHUNK = 80
PAD = NCHUNK * CH - EPT
NP = 10112
RPT = NP // NS

_mesh = plsc.VectorSubcoreMesh(core_axis_name="c", subcore_axis_name="s")
_f32 = jnp.float32


@functools.partial(
    pl.kernel,
    out_type=jax.ShapeDtypeStruct((4 * NP, D), _f32),
    mesh=_mesh,
    scratch_types=[
        pltpu.VMEM_SHARED((NP, D), _f32),
        pltpu.VMEM((CH,), jnp.int32),
        pltpu.VMEM((CH,), jnp.int32),
        pltpu.VMEM((CH, D), _f32),
    ],
)
def _deg_kernel(srch_hbm, dsth_hbm, zf_hbm, ones_hbm, hout,
                acc, sidx, didx, ones_v):
    c = lax.axis_index("c")
    s = lax.axis_index("s")
    w = c * NS + s
    r0 = s * RPT
    pltpu.sync_copy(zf_hbm.at[pl.ds(r0, RPT)], acc.at[pl.ds(r0, RPT)])
    pltpu.sync_copy(ones_hbm, ones_v)
    plsc.subcore_barrier()

    @pl.loop(0, NCHUNK)
    def _(j):
        pltpu.sync_copy(srch_hbm.at[w, j], sidx)
        pltpu.sync_copy(ones_v, acc.at[sidx], add=True)

    plsc.subcore_barrier()
    pltpu.sync_copy(acc.at[pl.ds(r0, RPT)], hout.at[pl.ds(c * NP + r0, RPT)])
    pltpu.sync_copy(zf_hbm.at[pl.ds(r0, RPT)], acc.at[pl.ds(r0, RPT)])
    plsc.subcore_barrier()

    @pl.loop(0, NCHUNK)
    def _(j):
        pltpu.sync_copy(dsth_hbm.at[w, j], didx)
        pltpu.sync_copy(ones_v, acc.at[didx], add=True)

    plsc.subcore_barrier()
    pltpu.sync_copy(acc.at[pl.ds(r0, RPT)],
                    hout.at[pl.ds((2 + c) * NP + r0, RPT)])


@functools.partial(
    pl.kernel,
    out_type=jax.ShapeDtypeStruct((2 * NP, D), _f32),
    mesh=_mesh,
    scratch_types=[
        pltpu.VMEM_SHARED((NP, D), _f32),
        pltpu.VMEM((CH,), jnp.int32),
        pltpu.VMEM((CH,), jnp.int32),
        pltpu.VMEM((CH, D), _f32),
    ],
)
def _agg_kernel(x_hbm, srcg_hbm, dsts_hbm, zf_hbm, pout,
                acc, sidx, didx, rows):
    c = lax.axis_index("c")
    s = lax.axis_index("s")
    w = c * NS + s
    r0 = s * RPT
    pltpu.sync_copy(zf_hbm.at[pl.ds(r0, RPT)], acc.at[pl.ds(r0, RPT)])
    plsc.subcore_barrier()

    @pl.loop(0, NCHUNK)
    def _(j):
        pltpu.sync_copy(srcg_hbm.at[w, j], sidx)
        pltpu.sync_copy(dsts_hbm.at[w, j], didx)
        pltpu.sync_copy(x_hbm.at[sidx], rows)
        pltpu.sync_copy(rows, acc.at[didx], add=True)

    plsc.subcore_barrier()
    pltpu.sync_copy(acc.at[pl.ds(r0, RPT)], pout.at[pl.ds(c * NP + r0, RPT)])


_BLK = 1000


def _scale_body(x_ref, s0_ref, s1_ref, d0_ref, d1_ref,
                xs_ref, do_ref, di_ref):
    dego = s0_ref[:, 0:1] + s1_ref[:, 0:1]
    degi = d0_ref[:, 0:1] + d1_ref[:, 0:1]
    do = lax.rsqrt(jnp.maximum(dego, 1.0))
    di = lax.rsqrt(jnp.maximum(degi, 1.0))
    do_b = jnp.broadcast_to(do, (_BLK, D))
    di_b = jnp.broadcast_to(di, (_BLK, D))
    xs_ref[...] = x_ref[...] * do_b
    do_ref[...] = do_b
    di_ref[...] = di_b


def _mm_body(p0_ref, p1_ref, di_ref, do_ref, w_ref, b_ref, o_ref, *, final):
    a = (p0_ref[...] + p1_ref[...]) * di_ref[...]
    z = lax.dot_general(a, w_ref[...], (((1,), (0,)), ((), ())),
                        precision=lax.Precision.HIGHEST,
                        preferred_element_type=_f32)
    z = z + b_ref[...]
    if final:
        o_ref[...] = z
    else:
        o_ref[...] = jnp.maximum(z, 0.0) * do_ref[...]


_spec_feat = pl.BlockSpec((_BLK, D), lambda i: (i, 0))
_spec_w = pl.BlockSpec((D, D), lambda i: (0, 0))
_spec_b = pl.BlockSpec((1, D), lambda i: (0, 0))

_scale_call = pl.pallas_call(
    _scale_body,
    grid=(N // _BLK,),
    in_specs=[_spec_feat] * 5,
    out_specs=[_spec_feat] * 3,
    out_shape=[jax.ShapeDtypeStruct((N, D), _f32) for _ in range(3)],
)

_mm_mid_call = pl.pallas_call(
    functools.partial(_mm_body, final=False),
    grid=(N // _BLK,),
    in_specs=[_spec_feat] * 4 + [_spec_w, _spec_b],
    out_specs=_spec_feat,
    out_shape=jax.ShapeDtypeStruct((N, D), _f32),
)

_mm_final_call = pl.pallas_call(
    functools.partial(_mm_body, final=True),
    grid=(N // _BLK,),
    in_specs=[_spec_feat] * 4 + [_spec_w, _spec_b],
    out_specs=_spec_feat,
    out_shape=jax.ShapeDtypeStruct((N, D), _f32),
)


def kernel(features, edge_index, W1, b1, W2, b2):
    src = edge_index[0].astype(jnp.int32).reshape(NW, EPT)
    dst = edge_index[1].astype(jnp.int32).reshape(NW, EPT)

    pad_valid = jnp.broadcast_to((jnp.arange(PAD, dtype=jnp.int32) * 61) % N,
                                 (NW, PAD))
    pad_dummy = jnp.broadcast_to(N + (jnp.arange(PAD, dtype=jnp.int32) % (NP - N)),
                                 (NW, PAD))
    src_g = jnp.concatenate([src, pad_valid], 1).reshape(NW, NCHUNK, CH)
    src_h = jnp.concatenate([src, pad_dummy], 1).reshape(NW, NCHUNK, CH)
    dst_s = jnp.concatenate([dst, pad_dummy], 1).reshape(NW, NCHUNK, CH)

    zeros_f = jnp.zeros((NP, D), _f32)
    ones_f = jnp.ones((CH, D), _f32)

    hist = _deg_kernel(src_h, dst_s, zeros_f, ones_f)
    s0, s1 = hist[0:N], hist[NP:NP + N]
    d0, d1 = hist[2 * NP:2 * NP + N], hist[3 * NP:3 * NP + N]
    xs, do_b, di_b = _scale_call(features, s0, s1, d0, d1)

    p = _agg_kernel(xs, src_g, dst_s, zeros_f)
    h1 = _mm_mid_call(p[0:N], p[NP:NP + N], di_b, do_b, W1, b1.reshape(1, D))

    q = _agg_kernel(h1, src_g, dst_s, zeros_f)
    out = _mm_final_call(q[0:N], q[NP:NP + N], di_b, do_b, W2, b2.reshape(1, D))
    return out


# speedup vs baseline: 8.6517x; 1.7587x over previous
import functools

import jax
import jax.numpy as jnp
from jax import lax
from jax.experimental import pallas as pl
from jax.experimental.pallas import tpu as pltpu
from jax.experimental.pallas import tpu_sc as plsc

N = 10000
E = 320000
D = 128
NC = 2
NS = 16
NW = NC * NS
EPT = E // NW
CH = 128
NCHUNK = 80
PAD = NCHUNK * CH - EPT
NP = 10112
RPT = NP // NS

_mesh = plsc.VectorSubcoreMesh(core_axis_name="c", subcore_axis_name="s")
_f32 = jnp.float32


_DRAIN = 8


@functools.partial(
    pl.kernel,
    out_type=jax.ShapeDtypeStruct((4 * NP, D), _f32),
    mesh=_mesh,
    scratch_types=[
        pltpu.VMEM_SHARED((NP, D), _f32),
        pltpu.VMEM((NCHUNK, CH), jnp.int32),
        pltpu.VMEM((NCHUNK, CH), jnp.int32),
        pltpu.VMEM((CH, D), _f32),
        pltpu.SemaphoreType.DMA,
    ],
)
def _deg_kernel(srch_hbm, dsth_hbm, zf_hbm, ones_hbm, hout,
                acc, sidx, didx, ones_v, sem):
    c = lax.axis_index("c")
    s = lax.axis_index("s")
    w = c * NS + s
    r0 = s * RPT
    pltpu.sync_copy(zf_hbm.at[pl.ds(r0, RPT)], acc.at[pl.ds(r0, RPT)])
    pltpu.sync_copy(srch_hbm.at[w], sidx)
    pltpu.sync_copy(dsth_hbm.at[w], didx)
    pltpu.sync_copy(ones_hbm, ones_v)
    plsc.subcore_barrier()

    def _hist_phase(idx):
        @pl.loop(0, NCHUNK)
        def _(j):
            @pl.when(j >= _DRAIN)
            def _():
                pltpu.make_async_copy(ones_v, acc.at[idx.at[0]], sem).wait()
            pltpu.async_copy(ones_v, acc.at[idx.at[j]], sem, add=True)

        @pl.loop(0, _DRAIN)
        def _(j):
            pltpu.make_async_copy(ones_v, acc.at[idx.at[0]], sem).wait()

    _hist_phase(sidx)
    plsc.subcore_barrier()
    pltpu.sync_copy(acc.at[pl.ds(r0, RPT)], hout.at[pl.ds(c * NP + r0, RPT)])
    pltpu.sync_copy(zf_hbm.at[pl.ds(r0, RPT)], acc.at[pl.ds(r0, RPT)])
    plsc.subcore_barrier()
    _hist_phase(didx)
    plsc.subcore_barrier()
    pltpu.sync_copy(acc.at[pl.ds(r0, RPT)],
                    hout.at[pl.ds((2 + c) * NP + r0, RPT)])


_NBUF = 2
_NIDX = 4


@functools.partial(
    pl.kernel,
    out_type=jax.ShapeDtypeStruct((2 * NP, D), _f32),
    mesh=_mesh,
    scratch_types=[
        pltpu.VMEM_SHARED((NP, D), _f32),
        pltpu.VMEM((_NIDX, 2, CH), jnp.int32),
        pltpu.VMEM((_NBUF, CH, D), _f32),
        pltpu.SemaphoreType.DMA((_NIDX,)),
        pltpu.SemaphoreType.DMA((_NBUF,)),
        pltpu.SemaphoreType.DMA((_NBUF,)),
    ],
)
def _agg_kernel(x_hbm, eidx_hbm, zf_hbm, pout,
                acc, ibuf, rows, isem, gsem, ssem):
    c = lax.axis_index("c")
    s = lax.axis_index("s")
    w = c * NS + s
    r0 = s * RPT
    pltpu.sync_copy(zf_hbm.at[pl.ds(r0, RPT)], acc.at[pl.ds(r0, RPT)])
    plsc.subcore_barrier()

    def _idx_fetch(j, q):
        pltpu.async_copy(eidx_hbm.at[w, j], ibuf.at[q], isem.at[q])

    def _wait_idx(q):
        pltpu.make_async_copy(eidx_hbm.at[w, 0], ibuf.at[q],
                              isem.at[q]).wait()

    def _wait_scat(t):
        pltpu.make_async_copy(rows.at[t], acc.at[ibuf.at[0, 1]],
                              ssem.at[t]).wait()

    def _wait_gath(t):
        pltpu.make_async_copy(x_hbm.at[ibuf.at[0, 0]], rows.at[t],
                              gsem.at[t]).wait()

    def _scat(j, t, q):
        pltpu.async_copy(rows.at[t], acc.at[ibuf.at[q, 1]],
                         ssem.at[t], add=True)

    _idx_fetch(0, 0)
    _idx_fetch(1, 1)

    @pl.loop(0, NCHUNK, step=_NIDX)
    def _(g):
        for b in range(_NIDX):
            j = g + b
            t = b % _NBUF
            q = b

            @pl.when(j >= _NBUF)
            def _():
                _wait_scat(t)

            @pl.when(j + 2 < NCHUNK)
            def _():
                _idx_fetch(j + 2, (b + 2) % _NIDX)
            _wait_idx(q)
            pltpu.async_copy(x_hbm.at[ibuf.at[q, 0]], rows.at[t], gsem.at[t])

            t1 = (b - 1) % _NBUF
            q1 = (b - 1) % _NIDX

            @pl.when(j >= 1)
            def _():
                _wait_gath(t1)
                _scat(j - 1, t1, q1)

    _wait_gath((NCHUNK - 1) % _NBUF)
    _scat(NCHUNK - 1, (NCHUNK - 1) % _NBUF, (NCHUNK - 1) % _NIDX)
    for t in range(_NBUF):
        _wait_scat(t)

    plsc.subcore_barrier()
    pltpu.sync_copy(acc.at[pl.ds(r0, RPT)], pout.at[pl.ds(c * NP + r0, RPT)])


_BLK = 1000


def _scale_body(x_ref, s0_ref, s1_ref, d0_ref, d1_ref,
                xs_ref, do_ref, di_ref):
    dego = s0_ref[:, 0:1] + s1_ref[:, 0:1]
    degi = d0_ref[:, 0:1] + d1_ref[:, 0:1]
    do = lax.rsqrt(jnp.maximum(dego, 1.0))
    di = lax.rsqrt(jnp.maximum(degi, 1.0))
    do_b = jnp.broadcast_to(do, (_BLK, D))
    di_b = jnp.broadcast_to(di, (_BLK, D))
    xs_ref[...] = x_ref[...] * do_b
    do_ref[...] = do_b
    di_ref[...] = di_b


def _mm_body(p0_ref, p1_ref, di_ref, do_ref, w_ref, b_ref, o_ref, *, final):
    a = (p0_ref[...] + p1_ref[...]) * di_ref[...]
    z = lax.dot_general(a, w_ref[...], (((1,), (0,)), ((), ())),
                        precision=lax.Precision.HIGHEST,
                        preferred_element_type=_f32)
    z = z + b_ref[...]
    if final:
        o_ref[...] = z
    else:
        o_ref[...] = jnp.maximum(z, 0.0) * do_ref[...]


_spec_feat = pl.BlockSpec((_BLK, D), lambda i: (i, 0))
_spec_w = pl.BlockSpec((D, D), lambda i: (0, 0))
_spec_b = pl.BlockSpec((1, D), lambda i: (0, 0))

_scale_call = pl.pallas_call(
    _scale_body,
    grid=(N // _BLK,),
    in_specs=[_spec_feat] * 5,
    out_specs=[_spec_feat] * 3,
    out_shape=[jax.ShapeDtypeStruct((N, D), _f32) for _ in range(3)],
)

_mm_mid_call = pl.pallas_call(
    functools.partial(_mm_body, final=False),
    grid=(N // _BLK,),
    in_specs=[_spec_feat] * 4 + [_spec_w, _spec_b],
    out_specs=_spec_feat,
    out_shape=jax.ShapeDtypeStruct((N, D), _f32),
)

_mm_final_call = pl.pallas_call(
    functools.partial(_mm_body, final=True),
    grid=(N // _BLK,),
    in_specs=[_spec_feat] * 4 + [_spec_w, _spec_b],
    out_specs=_spec_feat,
    out_shape=jax.ShapeDtypeStruct((N, D), _f32),
)


def kernel(features, edge_index, W1, b1, W2, b2):
    src = edge_index[0].astype(jnp.int32).reshape(NW, EPT)
    dst = edge_index[1].astype(jnp.int32).reshape(NW, EPT)

    pad_valid = jnp.broadcast_to((jnp.arange(PAD, dtype=jnp.int32) * 61) % N,
                                 (NW, PAD))
    pad_dummy = jnp.broadcast_to(N + (jnp.arange(PAD, dtype=jnp.int32) % (NP - N)),
                                 (NW, PAD))
    src_g = jnp.concatenate([src, pad_valid], 1).reshape(NW, NCHUNK, CH)
    src_h = jnp.concatenate([src, pad_dummy], 1).reshape(NW, NCHUNK, CH)
    dst_s = jnp.concatenate([dst, pad_dummy], 1).reshape(NW, NCHUNK, CH)
    eidx = jnp.stack([src_g, dst_s], axis=2)

    zeros_f = jnp.zeros((NP, D), _f32)
    ones_f = jnp.ones((CH, D), _f32)

    hist = _deg_kernel(src_h, dst_s, zeros_f, ones_f)
    s0, s1 = hist[0:N], hist[NP:NP + N]
    d0, d1 = hist[2 * NP:2 * NP + N], hist[3 * NP:3 * NP + N]
    xs, do_b, di_b = _scale_call(features, s0, s1, d0, d1)

    p = _agg_kernel(xs, eidx, zeros_f)
    h1 = _mm_mid_call(p[0:N], p[NP:NP + N], di_b, do_b, W1, b1.reshape(1, D))

    q = _agg_kernel(h1, eidx, zeros_f)
    out = _mm_final_call(q[0:N], q[NP:NP + N], di_b, do_b, W2, b2.reshape(1, D))
    return out


# speedup vs baseline: 9.1909x; 1.0623x over previous
import functools

import jax
import jax.numpy as jnp
from jax import lax
from jax.experimental import pallas as pl
from jax.experimental.pallas import tpu as pltpu
from jax.experimental.pallas import tpu_sc as plsc

N = 10000
E = 320000
D = 128
NC = 2
NS = 16
NW = NC * NS
EPT = E // NW
CH = 128
NCHUNK = 80
PAD = NCHUNK * CH - EPT
NP = 10112
RPT = NP // NS

_mesh = plsc.VectorSubcoreMesh(core_axis_name="c", subcore_axis_name="s")
_f32 = jnp.float32


_DRAIN = 8


_NIDX = 4


@functools.partial(
    pl.kernel,
    out_type=jax.ShapeDtypeStruct((NC, NP, D), _f32),
    mesh=_mesh,
    scratch_types=[
        pltpu.VMEM_SHARED((NP, D), _f32),
        pltpu.VMEM((_NIDX, 2, CH), jnp.int32),
        pltpu.VMEM((2, CH, D), _f32),
        pltpu.SemaphoreType.DMA((_NIDX,)),
        pltpu.SemaphoreType.DMA,
    ],
)
def _deg_kernel(eidxh_hbm, zf_hbm, ones_hbm, hout,
                acc, ibuf, ones_v, isem, sem):
    c = lax.axis_index("c")
    s = lax.axis_index("s")
    w = c * NS + s
    r0 = s * RPT
    pltpu.sync_copy(zf_hbm.at[pl.ds(r0, RPT)], acc.at[pl.ds(r0, RPT)])
    pltpu.sync_copy(ones_hbm, ones_v)
    plsc.subcore_barrier()

    def _idx_fetch(j, q):
        pltpu.async_copy(eidxh_hbm.at[w, j], ibuf.at[q], isem.at[q])

    def _wait_idx(q):
        pltpu.make_async_copy(eidxh_hbm.at[w, 0], ibuf.at[q],
                              isem.at[q]).wait()

    def _drain_scat():
        pltpu.make_async_copy(ones_v.at[0], acc.at[ibuf.at[0, 0]],
                              sem).wait()

    _idx_fetch(0, 0)
    _idx_fetch(1, 1)

    @pl.loop(0, NCHUNK, step=_NIDX)
    def _(g):
        for b in range(_NIDX):
            j = g + b
            q = b

            @pl.when(j >= 2)
            def _():
                _drain_scat()
                _drain_scat()

            @pl.when(j + 2 < NCHUNK)
            def _():
                _idx_fetch(j + 2, (b + 2) % _NIDX)
            _wait_idx(q)
            pltpu.async_copy(ones_v.at[0], acc.at[ibuf.at[q, 0]],
                             sem, add=True)
            pltpu.async_copy(ones_v.at[1], acc.at[ibuf.at[q, 1]],
                             sem, add=True)

    for _t in range(4):
        _drain_scat()

    plsc.subcore_barrier()
    pltpu.sync_copy(acc.at[pl.ds(r0, RPT)], hout.at[c, pl.ds(r0, RPT)])


_NBUF = 2
_NIDX = 4


@functools.partial(
    pl.kernel,
    out_type=jax.ShapeDtypeStruct((NC, NP, D), _f32),
    mesh=_mesh,
    scratch_types=[
        pltpu.VMEM_SHARED((NP, D), _f32),
        pltpu.VMEM((_NIDX, 2, CH), jnp.int32),
        pltpu.VMEM((_NBUF, CH, D), _f32),
        pltpu.SemaphoreType.DMA((_NIDX,)),
        pltpu.SemaphoreType.DMA((_NBUF,)),
        pltpu.SemaphoreType.DMA((_NBUF,)),
    ],
)
def _agg_kernel(x_hbm, eidx_hbm, zf_hbm, pout,
                acc, ibuf, rows, isem, gsem, ssem):
    c = lax.axis_index("c")
    s = lax.axis_index("s")
    w = c * NS + s
    r0 = s * RPT
    pltpu.sync_copy(zf_hbm.at[pl.ds(r0, RPT)], acc.at[pl.ds(r0, RPT)])
    plsc.subcore_barrier()

    def _idx_fetch(j, q):
        pltpu.async_copy(eidx_hbm.at[w, j], ibuf.at[q], isem.at[q])

    def _wait_idx(q):
        pltpu.make_async_copy(eidx_hbm.at[w, 0], ibuf.at[q],
                              isem.at[q]).wait()

    def _wait_scat(t):
        pltpu.make_async_copy(rows.at[t], acc.at[ibuf.at[0, 1]],
                              ssem.at[t]).wait()

    def _wait_gath(t):
        pltpu.make_async_copy(x_hbm.at[ibuf.at[0, 0]], rows.at[t],
                              gsem.at[t]).wait()

    def _scat(j, t, q):
        pltpu.async_copy(rows.at[t], acc.at[ibuf.at[q, 1]],
                         ssem.at[t], add=True)

    _idx_fetch(0, 0)
    _idx_fetch(1, 1)

    @pl.loop(0, NCHUNK, step=_NIDX)
    def _(g):
        for b in range(_NIDX):
            j = g + b
            t = b % _NBUF
            q = b

            @pl.when(j >= _NBUF)
            def _():
                _wait_scat(t)

            @pl.when(j + 2 < NCHUNK)
            def _():
                _idx_fetch(j + 2, (b + 2) % _NIDX)
            _wait_idx(q)
            pltpu.async_copy(x_hbm.at[ibuf.at[q, 0]], rows.at[t], gsem.at[t])

            t1 = (b - 1) % _NBUF
            q1 = (b - 1) % _NIDX

            @pl.when(j >= 1)
            def _():
                _wait_gath(t1)
                _scat(j - 1, t1, q1)

    _wait_gath((NCHUNK - 1) % _NBUF)
    _scat(NCHUNK - 1, (NCHUNK - 1) % _NBUF, (NCHUNK - 1) % _NIDX)
    for t in range(_NBUF):
        _wait_scat(t)

    plsc.subcore_barrier()
    pltpu.sync_copy(acc.at[pl.ds(r0, RPT)], pout.at[c, pl.ds(r0, RPT)])


_BLK = 1000


def _scale_body(x_ref, h0_ref, h1_ref, xs_ref, do_ref, di_ref):
    hsum = h0_ref[0] + h1_ref[0]
    dego = hsum[:, 0:1]
    degi = hsum[:, 64:65]
    do = lax.rsqrt(jnp.maximum(dego, 1.0))
    di = lax.rsqrt(jnp.maximum(degi, 1.0))
    do_b = jnp.broadcast_to(do, (_BLK, D))
    di_b = jnp.broadcast_to(di, (_BLK, D))
    xs_ref[...] = x_ref[...] * do_b
    do_ref[...] = do_b
    di_ref[...] = di_b


def _mm_body(p0_ref, p1_ref, di_ref, do_ref, w_ref, b_ref, o_ref, *, final):
    a = (p0_ref[0] + p1_ref[0]) * di_ref[...]
    z = lax.dot_general(a, w_ref[...], (((1,), (0,)), ((), ())),
                        precision=lax.Precision.HIGHEST,
                        preferred_element_type=_f32)
    z = z + b_ref[...]
    if final:
        o_ref[...] = z
    else:
        o_ref[...] = jnp.maximum(z, 0.0) * do_ref[...]


_spec_feat = pl.BlockSpec((_BLK, D), lambda i: (i, 0))
_spec_p0 = pl.BlockSpec((1, _BLK, D), lambda i: (0, i, 0))
_spec_p1 = pl.BlockSpec((1, _BLK, D), lambda i: (1, i, 0))
_spec_w = pl.BlockSpec((D, D), lambda i: (0, 0))
_spec_b = pl.BlockSpec((1, D), lambda i: (0, 0))

_scale_call = pl.pallas_call(
    _scale_body,
    grid=(N // _BLK,),
    in_specs=[_spec_feat, _spec_p0, _spec_p1],
    out_specs=[_spec_feat] * 3,
    out_shape=[jax.ShapeDtypeStruct((N, D), _f32) for _ in range(3)],
)

_mm_mid_call = pl.pallas_call(
    functools.partial(_mm_body, final=False),
    grid=(N // _BLK,),
    in_specs=[_spec_p0, _spec_p1, _spec_feat, _spec_feat, _spec_w, _spec_b],
    out_specs=_spec_feat,
    out_shape=jax.ShapeDtypeStruct((N, D), _f32),
)

_mm_final_call = pl.pallas_call(
    functools.partial(_mm_body, final=True),
    grid=(N // _BLK,),
    in_specs=[_spec_p0, _spec_p1, _spec_feat, _spec_feat, _spec_w, _spec_b],
    out_specs=_spec_feat,
    out_shape=jax.ShapeDtypeStruct((N, D), _f32),
)


def kernel(features, edge_index, W1, b1, W2, b2):
    src = edge_index[0].astype(jnp.int32).reshape(NW, EPT)
    dst = edge_index[1].astype(jnp.int32).reshape(NW, EPT)

    pad_valid = jnp.broadcast_to((jnp.arange(PAD, dtype=jnp.int32) * 61) % N,
                                 (NW, PAD))
    pad_dummy = jnp.broadcast_to(N + (jnp.arange(PAD, dtype=jnp.int32) % (NP - N)),
                                 (NW, PAD))
    src_g = jnp.concatenate([src, pad_valid], 1).reshape(NW, NCHUNK, CH)
    src_h = jnp.concatenate([src, pad_dummy], 1).reshape(NW, NCHUNK, CH)
    dst_s = jnp.concatenate([dst, pad_dummy], 1).reshape(NW, NCHUNK, CH)
    eidx = jnp.stack([src_g, dst_s], axis=2)
    eidx_h = jnp.stack([src_h, dst_s], axis=2)

    zeros_f = jnp.zeros((NP, D), _f32)
    lane = jnp.arange(D, dtype=jnp.int32)
    ones_f = jnp.stack([jnp.broadcast_to((lane < 64).astype(_f32), (CH, D)),
                        jnp.broadcast_to((lane >= 64).astype(_f32), (CH, D))])

    hist = _deg_kernel(eidx_h, zeros_f, ones_f)
    xs, do_b, di_b = _scale_call(features, hist, hist)

    p = _agg_kernel(xs, eidx, zeros_f)
    h1 = _mm_mid_call(p, p, di_b, do_b, W1, b1.reshape(1, D))

    q = _agg_kernel(h1, eidx, zeros_f)
    out = _mm_final_call(q, q, di_b, do_b, W2, b2.reshape(1, D))
    return out
